# Initial kernel scaffold; baseline (speedup 1.0000x reference)
import jax
import jax.numpy as jnp
from jax.experimental import pallas as pl


def kernel(x, W1, b1, W2, b2, W3, b3, meta_W, meta_b):
    raise NotImplementedError("write your pallas kernel here")


# speedup vs baseline: 1.4528x; 1.4528x over previous
import jax
import jax.numpy as jnp
from jax.experimental import pallas as pl

_N = 2048
_R = 256


def _layer_body(x_ref, w_ref, b_ref, mw_ref, mb_ref, out_w_ref, act_ref):
    pid = pl.program_id(0)
    w = w_ref[...]
    x = x_ref[...]
    y = jnp.sum(w * x, axis=1)
    a = jnp.maximum(y + b_ref[...], 0.0)
    act_ref[...] = a[None, :]

    @pl.when(pid == 0)
    def _():
        a0 = jnp.sum(jnp.where(jax.lax.iota(jnp.int32, _R) == 0, a, 0.0))
        patch = (mw_ref[0, 0] * x_ref[0, 0]
                 + mw_ref[0, 1] * w_ref[0, 0]
                 + mw_ref[0, 2] * a0
                 + mb_ref[0])
        rows = jax.lax.broadcasted_iota(jnp.int32, (_R, _N), 0)
        cols = jax.lax.broadcasted_iota(jnp.int32, (_R, _N), 1)
        out_w_ref[...] = jnp.where((rows == 0) & (cols == 0), patch, w)

    @pl.when(pid != 0)
    def _():
        out_w_ref[...] = w


def _layer(x, w, b, mw, mb, interpret=False):
    nb = _N // _R
    return pl.pallas_call(
        _layer_body,
        grid=(nb,),
        in_specs=[
            pl.BlockSpec((1, _N), lambda i: (0, 0)),
            pl.BlockSpec((_R, _N), lambda i: (i, 0)),
            pl.BlockSpec((_R,), lambda i: (i,)),
            pl.BlockSpec((1, 3), lambda i: (0, 0)),
            pl.BlockSpec((1,), lambda i: (0,)),
        ],
        out_specs=[
            pl.BlockSpec((_R, _N), lambda i: (i, 0)),
            pl.BlockSpec((1, _R), lambda i: (0, i)),
        ],
        out_shape=[
            jax.ShapeDtypeStruct((_N, _N), jnp.float32),
            jax.ShapeDtypeStruct((1, _N), jnp.float32),
        ],
        interpret=interpret,
    )(x, w, b, mw, mb)


def kernel(x, W1, b1, W2, b2, W3, b3, meta_W, meta_b):
    nw1, h1 = _layer(x, W1, b1, meta_W, meta_b)
    nw2, h2 = _layer(h1, W2, b2, meta_W, meta_b)
    nw3, out = _layer(h2, W3, b3, meta_W, meta_b)
    return out, nw1, nw2, nw3


# speedup vs baseline: 1.5042x; 1.0354x over previous
import jax
import jax.numpy as jnp
from jax.experimental import pallas as pl

_N = 2048
_R = 512


def _layer_body(x_ref, w_ref, b_ref, mw_ref, mb_ref, out_w_ref, act_ref):
    pid = pl.program_id(0)
    w = w_ref[...]
    x = x_ref[...]
    y = jnp.sum(w * x, axis=1)
    a = jnp.maximum(y + b_ref[...], 0.0)
    act_ref[...] = a[None, :]

    @pl.when(pid == 0)
    def _():
        a0 = jnp.sum(jnp.where(jax.lax.iota(jnp.int32, _R) == 0, a, 0.0))
        patch = (mw_ref[0, 0] * x_ref[0, 0]
                 + mw_ref[0, 1] * w_ref[0, 0]
                 + mw_ref[0, 2] * a0
                 + mb_ref[0])
        rows = jax.lax.broadcasted_iota(jnp.int32, (_R, _N), 0)
        cols = jax.lax.broadcasted_iota(jnp.int32, (_R, _N), 1)
        out_w_ref[...] = jnp.where((rows == 0) & (cols == 0), patch, w)

    @pl.when(pid != 0)
    def _():
        out_w_ref[...] = w


def _layer(x, w, b, mw, mb, interpret=False):
    nb = _N // _R
    return pl.pallas_call(
        _layer_body,
        grid=(nb,),
        in_specs=[
            pl.BlockSpec((1, _N), lambda i: (0, 0)),
            pl.BlockSpec((_R, _N), lambda i: (i, 0)),
            pl.BlockSpec((_R,), lambda i: (i,)),
            pl.BlockSpec((1, 3), lambda i: (0, 0)),
            pl.BlockSpec((1,), lambda i: (0,)),
        ],
        out_specs=[
            pl.BlockSpec((_R, _N), lambda i: (i, 0)),
            pl.BlockSpec((1, _R), lambda i: (0, i)),
        ],
        out_shape=[
            jax.ShapeDtypeStruct((_N, _N), jnp.float32),
            jax.ShapeDtypeStruct((1, _N), jnp.float32),
        ],
        interpret=interpret,
    )(x, w, b, mw, mb)


def kernel(x, W1, b1, W2, b2, W3, b3, meta_W, meta_b):
    nw1, h1 = _layer(x, W1, b1, meta_W, meta_b)
    nw2, h2 = _layer(h1, W2, b2, meta_W, meta_b)
    nw3, out = _layer(h2, W3, b3, meta_W, meta_b)
    return out, nw1, nw2, nw3


# speedup vs baseline: 1.6462x; 1.0944x over previous
import jax
import jax.numpy as jnp
from jax.experimental import pallas as pl

_N = 2048
_R = 1024


def _layer_body(x_ref, w_ref, b_ref, mw_ref, mb_ref, out_w_ref, act_ref):
    pid = pl.program_id(0)
    w = w_ref[...]
    x = x_ref[...]
    y = jnp.sum(w * x, axis=1)
    a = jnp.maximum(y + b_ref[...], 0.0)
    act_ref[...] = a[None, :]

    @pl.when(pid == 0)
    def _():
        a0 = jnp.sum(jnp.where(jax.lax.iota(jnp.int32, _R) == 0, a, 0.0))
        patch = (mw_ref[0, 0] * x_ref[0, 0]
                 + mw_ref[0, 1] * w_ref[0, 0]
                 + mw_ref[0, 2] * a0
                 + mb_ref[0])
        rows = jax.lax.broadcasted_iota(jnp.int32, (_R, _N), 0)
        cols = jax.lax.broadcasted_iota(jnp.int32, (_R, _N), 1)
        out_w_ref[...] = jnp.where((rows == 0) & (cols == 0), patch, w)

    @pl.when(pid != 0)
    def _():
        out_w_ref[...] = w


def _layer(x, w, b, mw, mb, interpret=False):
    nb = _N // _R
    return pl.pallas_call(
        _layer_body,
        grid=(nb,),
        in_specs=[
            pl.BlockSpec((1, _N), lambda i: (0, 0)),
            pl.BlockSpec((_R, _N), lambda i: (i, 0)),
            pl.BlockSpec((_R,), lambda i: (i,)),
            pl.BlockSpec((1, 3), lambda i: (0, 0)),
            pl.BlockSpec((1,), lambda i: (0,)),
        ],
        out_specs=[
            pl.BlockSpec((_R, _N), lambda i: (i, 0)),
            pl.BlockSpec((1, _R), lambda i: (0, i)),
        ],
        out_shape=[
            jax.ShapeDtypeStruct((_N, _N), jnp.float32),
            jax.ShapeDtypeStruct((1, _N), jnp.float32),
        ],
        interpret=interpret,
    )(x, w, b, mw, mb)


def kernel(x, W1, b1, W2, b2, W3, b3, meta_W, meta_b):
    nw1, h1 = _layer(x, W1, b1, meta_W, meta_b)
    nw2, h2 = _layer(h1, W2, b2, meta_W, meta_b)
    nw3, out = _layer(h2, W3, b3, meta_W, meta_b)
    return out, nw1, nw2, nw3
